# baseline (speedup 1.0000x reference)
import jax
import jax.numpy as jnp
from jax.experimental import pallas as pl
from jax.experimental.pallas import tpu as pltpu

T = 2048
NB = 4
CHUNK = 1024
TOKENS = 32768
D = 768
N_TILES = TOKENS // T
N_CHUNKS = T // CHUNK


def _pipe_kernel(x_hbm, w_ref, b_ref, out_hbm, xbuf, obuf, insem, outsem):
    def in_copy(i):
        s = i % NB
        return pltpu.make_async_copy(
            x_hbm.at[pl.ds(i * T, T), :], xbuf.at[s], insem.at[s]
        )

    def out_copy(i, c):
        s = i % NB
        return pltpu.make_async_copy(
            obuf.at[s, pl.ds(c * CHUNK, CHUNK), :],
            out_hbm.at[pl.ds(i * T + c * CHUNK, CHUNK), :],
            outsem.at[s, c],
        )

    for i in range(NB):
        in_copy(i).start()
    for i in range(N_TILES):
        s = i % NB
        in_copy(i).wait()
        for c in range(N_CHUNKS):
            if i >= NB:
                out_copy(i - NB, c).wait()
            rows = pl.ds(c * CHUNK, CHUNK)
            logits = jax.lax.dot_general(
                xbuf[s, c * CHUNK:(c + 1) * CHUNK, :].astype(jnp.bfloat16),
                w_ref[...],
                dimension_numbers=(((1,), (1,)), ((), ())),
                preferred_element_type=jnp.float32,
            )
            e = jnp.exp(logits + b_ref[...])
            obuf[s, rows, :] = e * (1.0 / jnp.sum(e, axis=-1, keepdims=True))
            out_copy(i, c).start()
        if i + NB < N_TILES:
            in_copy(i + NB).start()
    for i in range(N_TILES - NB, N_TILES):
        for c in range(N_CHUNKS):
            out_copy(i, c).wait()


@jax.jit
def kernel(x, W_gate, b_gate):
    tokens, d_model = x.shape
    b2d = b_gate.reshape(1, d_model)
    w_bf16 = W_gate.astype(jnp.bfloat16)
    return pl.pallas_call(
        _pipe_kernel,
        in_specs=[
            pl.BlockSpec(memory_space=pl.ANY),
            pl.BlockSpec(memory_space=pltpu.MemorySpace.VMEM),
            pl.BlockSpec(memory_space=pltpu.MemorySpace.VMEM),
        ],
        out_specs=pl.BlockSpec(memory_space=pl.ANY),
        out_shape=jax.ShapeDtypeStruct((tokens, d_model), jnp.float32),
        scratch_shapes=[
            pltpu.VMEM((NB, T, D), jnp.float32),
            pltpu.VMEM((NB, T, D), jnp.float32),
            pltpu.SemaphoreType.DMA((NB,)),
            pltpu.SemaphoreType.DMA((NB, N_CHUNKS)),
        ],
    )(x, w_bf16, b2d)


# speedup vs baseline: 1.0540x; 1.0540x over previous
import jax
import jax.numpy as jnp
from jax.experimental import pallas as pl
from jax.experimental.pallas import tpu as pltpu

T = 2048
NB = 4
CHUNK = 1024
TOKENS = 32768
D = 768
N_TILES = TOKENS // T


def _pipe_kernel(x_hbm, w_ref, b_ref, out_hbm, xbuf, obuf, insem, outsem):
    def in_copy(i):
        s = i % NB
        return pltpu.make_async_copy(
            x_hbm.at[pl.ds(i * T, T), :], xbuf.at[s], insem.at[s]
        )

    def out_copy(i):
        s = i % NB
        return pltpu.make_async_copy(
            obuf.at[s], out_hbm.at[pl.ds(i * T, T), :], outsem.at[s]
        )

    for i in range(NB):
        in_copy(i).start()
    for i in range(N_TILES):
        s = i % NB
        in_copy(i).wait()
        if i >= NB:
            out_copy(i - NB).wait()
        for c in range(T // CHUNK):
            rows = pl.ds(c * CHUNK, CHUNK)
            logits = jax.lax.dot_general(
                xbuf[s, c * CHUNK:(c + 1) * CHUNK, :].astype(jnp.bfloat16),
                w_ref[...],
                dimension_numbers=(((1,), (1,)), ((), ())),
                preferred_element_type=jnp.float32,
            )
            e = jnp.exp(logits + b_ref[...])
            obuf[s, rows, :] = e * (1.0 / jnp.sum(e, axis=-1, keepdims=True))
        out_copy(i).start()
        if i + NB < N_TILES:
            in_copy(i + NB).start()
    for i in range(N_TILES - NB, N_TILES):
        out_copy(i).wait()


@jax.jit
def kernel(x, W_gate, b_gate):
    tokens, d_model = x.shape
    b2d = b_gate.reshape(1, d_model)
    w_bf16 = W_gate.astype(jnp.bfloat16)
    return pl.pallas_call(
        _pipe_kernel,
        in_specs=[
            pl.BlockSpec(memory_space=pl.ANY),
            pl.BlockSpec(memory_space=pltpu.MemorySpace.VMEM),
            pl.BlockSpec(memory_space=pltpu.MemorySpace.VMEM),
        ],
        out_specs=pl.BlockSpec(memory_space=pl.ANY),
        out_shape=jax.ShapeDtypeStruct((tokens, d_model), jnp.float32),
        scratch_shapes=[
            pltpu.VMEM((NB, T, D), jnp.float32),
            pltpu.VMEM((NB, T, D), jnp.float32),
            pltpu.SemaphoreType.DMA((NB,)),
            pltpu.SemaphoreType.DMA((NB,)),
        ],
    )(x, w_bf16, b2d)


# speedup vs baseline: 1.0735x; 1.0184x over previous
import jax
import jax.numpy as jnp
from jax.experimental import pallas as pl
from jax.experimental.pallas import tpu as pltpu

T = 2048
NB = 4
CHUNK = 1024
FCHUNK = 512
TOKENS = 32768
D = 768
N_TILES = TOKENS // T
NF = T // FCHUNK


def _pipe_kernel(x_hbm, w_ref, b_ref, out_hbm, xbuf, obuf,
                 insem, outsem, insem0, outseml):
    def in_copy(i):
        s = i % NB
        return pltpu.make_async_copy(
            x_hbm.at[pl.ds(i * T, T), :], xbuf.at[s], insem.at[s]
        )

    def in0_copy(c):
        rows = pl.ds(c * FCHUNK, FCHUNK)
        return pltpu.make_async_copy(
            x_hbm.at[rows, :], xbuf.at[0, rows, :], insem0.at[c]
        )

    def out_copy(i):
        s = i % NB
        return pltpu.make_async_copy(
            obuf.at[s], out_hbm.at[pl.ds(i * T, T), :], outsem.at[s]
        )

    def outl_copy(c):
        s = (N_TILES - 1) % NB
        rows = pl.ds(c * FCHUNK, FCHUNK)
        return pltpu.make_async_copy(
            obuf.at[s, rows, :],
            out_hbm.at[pl.ds((N_TILES - 1) * T + c * FCHUNK, FCHUNK), :],
            outseml.at[c],
        )

    def compute(s, base, nrows):
        rows = pl.ds(base, nrows)
        logits = jax.lax.dot_general(
            xbuf[s, base:base + nrows, :].astype(jnp.bfloat16),
            w_ref[...],
            dimension_numbers=(((1,), (1,)), ((), ())),
            preferred_element_type=jnp.float32,
        )
        e = jnp.exp(logits + b_ref[...])
        obuf[s, rows, :] = e * (1.0 / jnp.sum(e, axis=-1, keepdims=True))

    for c in range(NF):
        in0_copy(c).start()
    for i in range(1, NB):
        in_copy(i).start()

    for c in range(NF):
        in0_copy(c).wait()
        compute(0, c * FCHUNK, FCHUNK)
    out_copy(0).start()
    in_copy(NB).start()

    for i in range(1, N_TILES - 1):
        s = i % NB
        in_copy(i).wait()
        if i >= NB:
            out_copy(i - NB).wait()
        for c in range(T // CHUNK):
            compute(s, c * CHUNK, CHUNK)
        out_copy(i).start()
        if i + NB < N_TILES:
            in_copy(i + NB).start()

    i = N_TILES - 1
    s = i % NB
    in_copy(i).wait()
    out_copy(i - NB).wait()
    for c in range(NF):
        compute(s, c * FCHUNK, FCHUNK)
        outl_copy(c).start()

    for i in range(N_TILES - NB, N_TILES - 1):
        out_copy(i).wait()
    for c in range(NF):
        outl_copy(c).wait()


@jax.jit
def kernel(x, W_gate, b_gate):
    tokens, d_model = x.shape
    b2d = b_gate.reshape(1, d_model)
    w_bf16 = W_gate.astype(jnp.bfloat16)
    return pl.pallas_call(
        _pipe_kernel,
        in_specs=[
            pl.BlockSpec(memory_space=pl.ANY),
            pl.BlockSpec(memory_space=pltpu.MemorySpace.VMEM),
            pl.BlockSpec(memory_space=pltpu.MemorySpace.VMEM),
        ],
        out_specs=pl.BlockSpec(memory_space=pl.ANY),
        out_shape=jax.ShapeDtypeStruct((tokens, d_model), jnp.float32),
        scratch_shapes=[
            pltpu.VMEM((NB, T, D), jnp.float32),
            pltpu.VMEM((NB, T, D), jnp.float32),
            pltpu.SemaphoreType.DMA((NB,)),
            pltpu.SemaphoreType.DMA((NB,)),
            pltpu.SemaphoreType.DMA((NF,)),
            pltpu.SemaphoreType.DMA((NF,)),
        ],
    )(x, w_bf16, b2d)
